# Initial kernel scaffold; baseline (speedup 1.0000x reference)
import jax
import jax.numpy as jnp
from jax.experimental import pallas as pl


def kernel(x, edge_index, edge_vals, W, b):
    raise NotImplementedError("write your pallas kernel here")


# speedup vs baseline: 5.9129x; 5.9129x over previous
import functools

import jax
import jax.numpy as jnp
from jax import lax
from jax.experimental import pallas as pl
from jax.experimental.pallas import tpu as pltpu
from jax.experimental.pallas import tpu_sc as plsc

N = 10000
E = 320000
D_IN = 128
D = 128

NC = 2
NS = 16
L = 16
NW = NC * NS

K = 50
BURST = 8
ET = E // NW
CHUNKS_T = ET // K
NBURSTS = CHUNKS_T // BURST
BE = BURST * K

NP = 10240
ROWS_T = NP // NS

BR = 400
CR = 256

_SPLAT_DNUMS = lax.GatherDimensionNumbers(
    offset_dims=(), collapsed_slice_dims=(0,), start_index_map=(0,))


def _lane_splat(vec, lane):
    idx = jnp.full((L, 1), lane, jnp.int32)
    return lax.gather(vec, idx, _SPLAT_DNUMS, (1,),
                      mode=lax.GatherScatterMode.PROMISE_IN_BOUNDS)


def _mm_body(x_ref, w_ref, o_ref):
    xc = jnp.clip(x_ref[...], -10.0, 10.0)
    s = jnp.dot(xc, w_ref[...], preferred_element_type=jnp.float32)
    o_ref[...] = jnp.clip(s, -10.0, 10.0)


def _support(x, W):
    return pl.pallas_call(
        _mm_body,
        grid=(N // BR,),
        in_specs=[
            pl.BlockSpec((BR, D_IN), lambda i: (i, 0)),
            pl.BlockSpec((D_IN, D), lambda i: (0, 0)),
        ],
        out_specs=pl.BlockSpec((BR, D), lambda i: (i, 0)),
        out_shape=jax.ShapeDtypeStruct((N, D), jnp.float32),
    )(x, W)


def _combine_body(p_ref, b_ref, o_ref):
    s = jnp.clip(p_ref[0] + p_ref[1], -10.0, 10.0) + b_ref[...]
    o_ref[...] = jnp.clip(jax.nn.relu(s), -10.0, 10.0)


def _combine(partials, b2d):
    return pl.pallas_call(
        _combine_body,
        grid=(NP // CR,),
        in_specs=[
            pl.BlockSpec((NC, CR, D), lambda i: (0, i, 0)),
            pl.BlockSpec((1, D), lambda i: (0, 0)),
        ],
        out_specs=pl.BlockSpec((CR, D), lambda i: (i, 0)),
        out_shape=jax.ShapeDtypeStruct((NP, D), jnp.float32),
    )(partials, b2d)


USCALE = 10


def _scale_chunk(buf, valv, j):
    def scale_body(it, carry):
        base = it * USCALE
        vv = valv[pl.ds(j * K + base, L)]
        for u in range(USCALE):
            e = base + u
            s = _lane_splat(vv, u)
            for q in range(D // L):
                v = buf[e, pl.ds(q * L, L)]
                buf[e, pl.ds(q * L, L)] = v * s
        return carry

    lax.fori_loop(0, K // USCALE, scale_body, 0)


def _sc_body(sup_hbm, src_hbm, dst_hbm, val_hbm, zeros_hbm,
             out_hbm, acc, srcv, dstv, valv, bufa, bufb, sem):
    c = lax.axis_index("c")
    t = lax.axis_index("s")
    w = c * NS + t

    pltpu.sync_copy(zeros_hbm, acc.at[pl.ds(t * ROWS_T, ROWS_T)])
    plsc.subcore_barrier()

    bufs = [bufa, bufb]

    def burst_body(g, carry):
        row0 = w * CHUNKS_T + g * BURST
        pltpu.sync_copy(src_hbm.at[pl.ds(row0, BURST)], srcv)
        pltpu.sync_copy(dst_hbm.at[pl.ds(row0, BURST)], dstv)
        pltpu.sync_copy(val_hbm.at[pl.ds(row0 * K, BE)],
                        valv.at[pl.ds(0, BE)])

        pltpu.async_copy(sup_hbm.at[srcv.at[0]], bufs[0], sem).wait()
        for j in range(BURST):
            if j + 1 < BURST:
                nxt = pltpu.async_copy(sup_hbm.at[srcv.at[j + 1]],
                                       bufs[(j + 1) % 2], sem)
            buf = bufs[j % 2]
            _scale_chunk(buf, valv, j)
            pltpu.sync_copy(buf, acc.at[dstv.at[j]], add=True)
            if j + 1 < BURST:
                nxt.wait()
        return carry

    lax.fori_loop(0, NBURSTS, burst_body, 0)
    plsc.subcore_barrier()

    r0 = t * ROWS_T
    pltpu.sync_copy(acc.at[pl.ds(r0, ROWS_T)],
                    out_hbm.at[c, pl.ds(r0, ROWS_T)])


_sc_spmm = pl.kernel(
    _sc_body,
    out_type=jax.ShapeDtypeStruct((NC, NP, D), jnp.float32),
    mesh=plsc.VectorSubcoreMesh(core_axis_name="c", subcore_axis_name="s",
                                num_cores=NC, num_subcores=NS),
    scratch_types=[
        pltpu.VMEM_SHARED((NP, D), jnp.float32),
        pltpu.VMEM((BURST, K), jnp.int32),
        pltpu.VMEM((BURST, K), jnp.int32),
        pltpu.VMEM((BE + L,), jnp.float32),
        pltpu.VMEM((K, D), jnp.float32),
        pltpu.VMEM((K, D), jnp.float32),
        pltpu.SemaphoreType.DMA,
    ],
)


@jax.jit
def _run(x, edge_index, edge_vals, W, b):
    sup = _support(x, W)
    dst = edge_index[0]
    src = edge_index[1]
    src2 = src.reshape(E // K, K)
    dst2 = dst.reshape(E // K, K)
    zeros = jnp.zeros((ROWS_T, D), jnp.float32)
    partials = _sc_spmm(sup, src2, dst2, edge_vals, zeros)
    out = _combine(partials, b[None])
    return out[:N]


def kernel(x, edge_index, edge_vals, W, b):
    return _run(x, edge_index, edge_vals, W, b)


# speedup vs baseline: 8.8348x; 1.4942x over previous
import functools

import jax
import jax.numpy as jnp
from jax import lax
from jax.experimental import pallas as pl
from jax.experimental.pallas import tpu as pltpu
from jax.experimental.pallas import tpu_sc as plsc

N = 10000
E = 320000
D_IN = 128
D = 128

NC = 2
NS = 16
L = 16
NW = NC * NS

K = 50
BURST = 40
NB = 4
ET = E // NW
CHUNKS_T = ET // K
NBURSTS = CHUNKS_T // BURST
BE = BURST * K

NP = 10240
ROWS_T = NP // NS

BR = 400
CR = 256

_SPLAT_DNUMS = lax.GatherDimensionNumbers(
    offset_dims=(), collapsed_slice_dims=(0,), start_index_map=(0,))


def _lane_splat(vec, lane):
    idx = jnp.full((L, 1), lane, jnp.int32)
    return lax.gather(vec, idx, _SPLAT_DNUMS, (1,),
                      mode=lax.GatherScatterMode.PROMISE_IN_BOUNDS)


def _mm_body(x_ref, w_ref, o_ref):
    xc = jnp.clip(x_ref[...], -10.0, 10.0)
    s = jnp.dot(xc, w_ref[...], preferred_element_type=jnp.float32)
    o_ref[...] = jnp.clip(s, -10.0, 10.0)


def _support(x, W):
    return pl.pallas_call(
        _mm_body,
        grid=(N // BR,),
        in_specs=[
            pl.BlockSpec((BR, D_IN), lambda i: (i, 0)),
            pl.BlockSpec((D_IN, D), lambda i: (0, 0)),
        ],
        out_specs=pl.BlockSpec((BR, D), lambda i: (i, 0)),
        out_shape=jax.ShapeDtypeStruct((N, D), jnp.float32),
    )(x, W)


def _combine_body(p_ref, b_ref, o_ref):
    s = jnp.clip(p_ref[0] + p_ref[1], -10.0, 10.0) + b_ref[...]
    o_ref[...] = jnp.clip(jax.nn.relu(s), -10.0, 10.0)


def _combine(partials, b2d):
    return pl.pallas_call(
        _combine_body,
        grid=(NP // CR,),
        in_specs=[
            pl.BlockSpec((NC, CR, D), lambda i: (0, i, 0)),
            pl.BlockSpec((1, D), lambda i: (0, 0)),
        ],
        out_specs=pl.BlockSpec((CR, D), lambda i: (i, 0)),
        out_shape=jax.ShapeDtypeStruct((NP, D), jnp.float32),
    )(partials, b2d)


USCALE = 10


def _scale_chunk(buf, valv, j):
    def scale_body(it, carry):
        base = it * USCALE
        vv = valv[pl.ds(j * K + base, L)]
        for u in range(USCALE):
            e = base + u
            s = _lane_splat(vv, u)
            for q in range(D // L):
                v = buf[e, pl.ds(q * L, L)]
                buf[e, pl.ds(q * L, L)] = v * s
        return carry

    lax.fori_loop(0, K // USCALE, scale_body, 0)


def _sc_body(sup_hbm, src_hbm, dst_hbm, val_hbm, zeros_hbm,
             out_hbm, acc, srcv, dstv, valv, bufa, bufb, bufc, bufd,
             gsem, ssem):
    c = lax.axis_index("c")
    t = lax.axis_index("s")
    w = c * NS + t

    pltpu.sync_copy(zeros_hbm, acc.at[pl.ds(t * ROWS_T, ROWS_T)])
    plsc.subcore_barrier()

    bufs = [bufa, bufb, bufc, bufd]

    def burst_body(g, carry):
        row0 = w * CHUNKS_T + g * BURST
        pltpu.sync_copy(src_hbm.at[pl.ds(row0, BURST)], srcv)
        pltpu.sync_copy(dst_hbm.at[pl.ds(row0, BURST)], dstv)
        pltpu.sync_copy(val_hbm.at[pl.ds(row0 * K, BE)],
                        valv.at[pl.ds(0, BE)])

        gw = [None] * BURST
        sw = [None] * BURST
        for j in range(min(3, BURST)):
            gw[j] = pltpu.async_copy(sup_hbm.at[srcv.at[j]], bufs[j % NB],
                                     gsem)
        for j in range(BURST):
            gw[j].wait()
            buf = bufs[j % NB]
            _scale_chunk(buf, valv, j)
            sw[j] = pltpu.async_copy(buf, acc.at[dstv.at[j]], ssem,
                                     add=True)
            if j >= 1:
                sw[j - 1].wait()
            if j + 3 < BURST:
                gw[j + 3] = pltpu.async_copy(sup_hbm.at[srcv.at[j + 3]],
                                             bufs[(j + 3) % NB], gsem)
        sw[BURST - 1].wait()
        return carry

    lax.fori_loop(0, NBURSTS, burst_body, 0)
    plsc.subcore_barrier()

    r0 = t * ROWS_T
    pltpu.sync_copy(acc.at[pl.ds(r0, ROWS_T)],
                    out_hbm.at[c, pl.ds(r0, ROWS_T)])


_sc_spmm = pl.kernel(
    _sc_body,
    out_type=jax.ShapeDtypeStruct((NC, NP, D), jnp.float32),
    mesh=plsc.VectorSubcoreMesh(core_axis_name="c", subcore_axis_name="s",
                                num_cores=NC, num_subcores=NS),
    scratch_types=[
        pltpu.VMEM_SHARED((NP, D), jnp.float32),
        pltpu.VMEM((BURST, K), jnp.int32),
        pltpu.VMEM((BURST, K), jnp.int32),
        pltpu.VMEM((BE + L,), jnp.float32),
        pltpu.VMEM((K, D), jnp.float32),
        pltpu.VMEM((K, D), jnp.float32),
        pltpu.VMEM((K, D), jnp.float32),
        pltpu.VMEM((K, D), jnp.float32),
        pltpu.SemaphoreType.DMA,
        pltpu.SemaphoreType.DMA,
    ],
)


@jax.jit
def _run(x, edge_index, edge_vals, W, b):
    sup = _support(x, W)
    dst = edge_index[0]
    src = edge_index[1]
    src2 = src.reshape(E // K, K)
    dst2 = dst.reshape(E // K, K)
    zeros = jnp.zeros((ROWS_T, D), jnp.float32)
    partials = _sc_spmm(sup, src2, dst2, edge_vals, zeros)
    out = _combine(partials, b[None])
    return out[:N]


def kernel(x, edge_index, edge_vals, W, b):
    return _run(x, edge_index, edge_vals, W, b)


# speedup vs baseline: 10.5150x; 1.1902x over previous
import functools

import jax
import jax.numpy as jnp
from jax import lax
from jax.experimental import pallas as pl
from jax.experimental.pallas import tpu as pltpu
from jax.experimental.pallas import tpu_sc as plsc

N = 10000
E = 320000
D_IN = 128
D = 128

NC = 2
NS = 16
L = 16
NW = NC * NS

K = 50
BURST = 40
NB = 4
ET = E // NW
CHUNKS_T = ET // K
NBURSTS = CHUNKS_T // BURST
BE = BURST * K

NP = 10112
ROWS_T = NP // NS

BR = 400
CR = 400

_SPLAT_DNUMS = lax.GatherDimensionNumbers(
    offset_dims=(), collapsed_slice_dims=(0,), start_index_map=(0,))


def _lane_splat(vec, lane):
    idx = jnp.full((L, 1), lane, jnp.int32)
    return lax.gather(vec, idx, _SPLAT_DNUMS, (1,),
                      mode=lax.GatherScatterMode.PROMISE_IN_BOUNDS)


def _mm_body(x_ref, w_ref, o_ref):
    xc = jnp.clip(x_ref[...], -10.0, 10.0)
    s = jnp.dot(xc, w_ref[...], preferred_element_type=jnp.float32)
    o_ref[...] = jnp.clip(s, -10.0, 10.0)


def _support(x, W):
    return pl.pallas_call(
        _mm_body,
        grid=(N // BR,),
        in_specs=[
            pl.BlockSpec((BR, D_IN), lambda i: (i, 0)),
            pl.BlockSpec((D_IN, D), lambda i: (0, 0)),
        ],
        out_specs=pl.BlockSpec((BR, D), lambda i: (i, 0)),
        out_shape=jax.ShapeDtypeStruct((N, D), jnp.float32),
    )(x, W)


def _combine_body(p_ref, b_ref, o_ref):
    s = jnp.clip(p_ref[0] + p_ref[1], -10.0, 10.0) + b_ref[...]
    o_ref[...] = jnp.clip(jax.nn.relu(s), -10.0, 10.0)


def _combine(partials, b2d):
    return pl.pallas_call(
        _combine_body,
        grid=(N // CR,),
        in_specs=[
            pl.BlockSpec((NC, CR, D), lambda i: (0, i, 0)),
            pl.BlockSpec((1, D), lambda i: (0, 0)),
        ],
        out_specs=pl.BlockSpec((CR, D), lambda i: (i, 0)),
        out_shape=jax.ShapeDtypeStruct((N, D), jnp.float32),
    )(partials, b2d)


USCALE = 10


def _scale_chunk(buf, valv, j):
    def scale_body(it, carry):
        base = it * USCALE
        vv = valv[pl.ds(j * K + base, L)]
        for u in range(USCALE):
            e = base + u
            s = _lane_splat(vv, u)
            for q in range(D // L):
                v = buf[e, pl.ds(q * L, L)]
                buf[e, pl.ds(q * L, L)] = v * s
        return carry

    lax.fori_loop(0, K // USCALE, scale_body, 0)


def _sc_body(sup_hbm, eix_hbm, val_hbm, zeros_hbm,
             out_hbm, acc, srcv, dstv, valv, bufa, bufb, bufc, bufd,
             gsem, ssem, isem, dvsem):
    c = lax.axis_index("c")
    t = lax.axis_index("s")
    w = c * NS + t

    bufs = [bufa, bufb, bufc, bufd]
    row00 = w * CHUNKS_T

    i_zero = pltpu.async_copy(zeros_hbm,
                              acc.at[pl.ds(t * ROWS_T, ROWS_T)], isem)
    i_src0 = pltpu.async_copy(eix_hbm.at[1, pl.ds(row00, BURST)],
                              srcv.at[pl.ds(0, BURST)], isem)
    pltpu.async_copy(eix_hbm.at[0, pl.ds(row00, BURST)], dstv, dvsem)
    pltpu.async_copy(val_hbm.at[pl.ds(row00 * K, BE)],
                     valv.at[pl.ds(0, BE)], dvsem)

    i_zero.wait()
    i_src0.wait()

    for j in range(min(3, BURST)):
        pltpu.async_copy(sup_hbm.at[srcv.at[j]], bufs[j % NB], gsem)

    plsc.subcore_barrier()

    def burst_body(g, carry):
        par = (g % 2) * BURST
        row0 = w * CHUNKS_T + g * BURST
        gn = (g + 1) % NBURSTS
        parn = ((g + 1) % 2) * BURST
        row0n = w * CHUNKS_T + gn * BURST

        i_srcn = pltpu.async_copy(eix_hbm.at[1, pl.ds(row0n, BURST)],
                                  srcv.at[pl.ds(parn, BURST)], isem)

        pltpu.make_async_copy(eix_hbm.at[0, pl.ds(row0, BURST)],
                              dstv, dvsem).wait()
        pltpu.make_async_copy(val_hbm.at[pl.ds(row0 * K, BE)],
                              valv.at[pl.ds(0, BE)], dvsem).wait()

        sw = [None] * BURST
        for j in range(BURST):
            pltpu.make_async_copy(sup_hbm.at[srcv.at[par + j]],
                                  bufs[j % NB], gsem).wait()
            buf = bufs[j % NB]
            _scale_chunk(buf, valv, j)
            sw[j] = pltpu.async_copy(buf, acc.at[dstv.at[j]], ssem,
                                     add=True)
            if j >= 1:
                sw[j - 1].wait()
            if j + 3 < BURST:
                pltpu.async_copy(sup_hbm.at[srcv.at[par + j + 3]],
                                 bufs[(j + 3) % NB], gsem)
            else:
                cjn = j + 3 - BURST
                if cjn == 0:
                    i_srcn.wait()
                pltpu.async_copy(sup_hbm.at[srcv.at[parn + cjn]],
                                 bufs[cjn % NB], gsem)
        sw[BURST - 1].wait()

        pltpu.async_copy(eix_hbm.at[0, pl.ds(row0n, BURST)], dstv, dvsem)
        pltpu.async_copy(val_hbm.at[pl.ds(row0n * K, BE)],
                         valv.at[pl.ds(0, BE)], dvsem)
        return carry

    lax.fori_loop(0, NBURSTS, burst_body, 0)

    pltpu.make_async_copy(eix_hbm.at[0, pl.ds(row00, BURST)],
                          dstv, dvsem).wait()
    pltpu.make_async_copy(val_hbm.at[pl.ds(row00 * K, BE)],
                          valv.at[pl.ds(0, BE)], dvsem).wait()

    for j in range(min(3, BURST)):
        pltpu.make_async_copy(sup_hbm.at[srcv.at[BURST + j]],
                              bufs[j % NB], gsem).wait()

    plsc.subcore_barrier()

    r0 = t * ROWS_T
    pltpu.sync_copy(acc.at[pl.ds(r0, ROWS_T)],
                    out_hbm.at[c, pl.ds(r0, ROWS_T)])


_sc_spmm = pl.kernel(
    _sc_body,
    out_type=jax.ShapeDtypeStruct((NC, NP, D), jnp.float32),
    mesh=plsc.VectorSubcoreMesh(core_axis_name="c", subcore_axis_name="s",
                                num_cores=NC, num_subcores=NS),
    scratch_types=[
        pltpu.VMEM_SHARED((NP, D), jnp.float32),
        pltpu.VMEM((2 * BURST, K), jnp.int32),
        pltpu.VMEM((BURST, K), jnp.int32),
        pltpu.VMEM((BE + L,), jnp.float32),
        pltpu.VMEM((K, D), jnp.float32),
        pltpu.VMEM((K, D), jnp.float32),
        pltpu.VMEM((K, D), jnp.float32),
        pltpu.VMEM((K, D), jnp.float32),
        pltpu.SemaphoreType.DMA,
        pltpu.SemaphoreType.DMA,
        pltpu.SemaphoreType.DMA,
        pltpu.SemaphoreType.DMA,
    ],
)


@jax.jit
def _run(x, edge_index, edge_vals, W, b):
    sup = _support(x, W)
    eix3 = edge_index.reshape(2, E // K, K)
    zeros = jnp.zeros((ROWS_T, D), jnp.float32)
    partials = _sc_spmm(sup, eix3, edge_vals, zeros)
    return _combine(partials, b[None])


def kernel(x, edge_index, edge_vals, W, b):
    return _run(x, edge_index, edge_vals, W, b)
